# baseline (speedup 1.0000x reference)
import jax
import jax.numpy as jnp
from jax.experimental import pallas as pl
from jax.experimental.pallas import tpu as pltpu

_D_IN = 768
_D_H = 1024
_NL = 4


def _matvec(x, w):
    return jax.lax.dot_general(
        x.astype(jnp.bfloat16), w.astype(jnp.bfloat16),
        (((1,), (1,)), ((), ())),
        preferred_element_type=jnp.float32)


def _body(q_ref, we_hbm, be_ref, wp_hbm, bp_ref, gp_ref, betap_ref,
          out_ref, we_v, wb0, wb1, sem_we, sem_w):
    wbufs = [wb0, wb1]
    cp_we = pltpu.make_async_copy(we_hbm, we_v, sem_we)
    cp_we.start()
    pltpu.make_async_copy(wp_hbm.at[0], wbufs[0], sem_w.at[0]).start()

    cp_we.wait()
    h = _matvec(q_ref[...], we_v[...]) + be_ref[...]
    n = jnp.sqrt(jnp.sum(h * h))
    x = h / jnp.maximum(n, 1e-12)

    for i in range(_NL):
        pltpu.make_async_copy(
            wp_hbm.at[i], wbufs[i % 2], sem_w.at[i % 2]).wait()
        if i + 1 < _NL:
            pltpu.make_async_copy(
                wp_hbm.at[i + 1], wbufs[(i + 1) % 2],
                sem_w.at[(i + 1) % 2]).start()
        h = _matvec(x, wbufs[i % 2][...]) + bp_ref[i, 0][None]
        h = 0.5 * h * (1.0 + jax.lax.erf(h * 0.7071067811865476))
        mu = jnp.mean(h, axis=-1, keepdims=True)
        var = jnp.mean((h - mu) * (h - mu), axis=-1, keepdims=True)
        h = (h - mu) / jnp.sqrt(var + 1e-5) * gp_ref[i, 0][None] \
            + betap_ref[i, 0][None]
        x = x + h

    out_ref[...] = x


def kernel(query, context, W_enc, b_enc, Wp, bp, gp, betap):
    del context
    q2 = query.reshape(1, _D_IN)
    be2 = b_enc.reshape(1, _D_H)
    out = pl.pallas_call(
        _body,
        in_specs=[
            pl.BlockSpec(memory_space=pltpu.MemorySpace.VMEM),
            pl.BlockSpec(memory_space=pltpu.MemorySpace.HBM),
            pl.BlockSpec(memory_space=pltpu.MemorySpace.VMEM),
            pl.BlockSpec(memory_space=pltpu.MemorySpace.HBM),
            pl.BlockSpec(memory_space=pltpu.MemorySpace.VMEM),
            pl.BlockSpec(memory_space=pltpu.MemorySpace.VMEM),
            pl.BlockSpec(memory_space=pltpu.MemorySpace.VMEM),
        ],
        out_specs=pl.BlockSpec(memory_space=pltpu.MemorySpace.VMEM),
        out_shape=jax.ShapeDtypeStruct((1, _D_H), jnp.float32),
        scratch_shapes=[
            pltpu.VMEM((_D_H, _D_IN), jnp.float32),
            pltpu.VMEM((_D_H, _D_H), jnp.float32),
            pltpu.VMEM((_D_H, _D_H), jnp.float32),
            pltpu.SemaphoreType.DMA,
            pltpu.SemaphoreType.DMA((2,)),
        ],
    )(q2, W_enc, be2, Wp, bp.reshape(_NL, 1, _D_H), gp.reshape(_NL, 1, _D_H),
      betap.reshape(_NL, 1, _D_H))
    return out.reshape(_D_H)


# speedup vs baseline: 1.1060x; 1.1060x over previous
import jax
import jax.numpy as jnp
from jax.experimental import pallas as pl
from jax.experimental.pallas import tpu as pltpu

_D_IN = 768
_D_H = 1024
_NL = 4


def _matvec(x, w):
    return jax.lax.dot_general(
        x.astype(jnp.bfloat16), w.astype(jnp.bfloat16),
        (((1,), (1,)), ((), ())),
        preferred_element_type=jnp.float32)


def _body(q_ref, we_hbm, be_ref, wp_hbm, bp_ref, gp_ref, betap_ref,
          out_ref, we_v, wb0, wb1, wb2, sem_we, sem_w):
    wbufs = [wb0, wb1, wb2]
    hh = _D_H // 2

    def _start(i):
        for s in range(2):
            pltpu.make_async_copy(
                wp_hbm.at[i, pl.ds(s * hh, hh)],
                wbufs[i % 3].at[pl.ds(s * hh, hh)],
                sem_w.at[i % 3, s]).start()

    def _wait(i, s):
        pltpu.make_async_copy(
            wp_hbm.at[i, pl.ds(s * hh, hh)],
            wbufs[i % 3].at[pl.ds(s * hh, hh)],
            sem_w.at[i % 3, s]).wait()

    cp_we = pltpu.make_async_copy(we_hbm, we_v, sem_we)
    cp_we.start()
    for i in range(2):
        _start(i)

    cp_we.wait()
    h = _matvec(q_ref[...], we_v[...]) + be_ref[...]
    n = jnp.sqrt(jnp.sum(h * h))
    x = h / jnp.maximum(n, 1e-12)

    for i in range(_NL):
        if i + 2 < _NL:
            _start(i + 2)
        _wait(i, 0)
        r0 = _matvec(x, wbufs[i % 3][pl.ds(0, hh), :])
        _wait(i, 1)
        r1 = _matvec(x, wbufs[i % 3][pl.ds(hh, hh), :])
        h = jnp.concatenate([r0, r1], axis=1) + bp_ref[i, 0][None]
        h = 0.5 * h * (1.0 + jax.lax.erf(h * 0.7071067811865476))
        mu = jnp.mean(h, axis=-1, keepdims=True)
        var = jnp.mean((h - mu) * (h - mu), axis=-1, keepdims=True)
        h = (h - mu) / jnp.sqrt(var + 1e-5) * gp_ref[i, 0][None] \
            + betap_ref[i, 0][None]
        x = x + h

    out_ref[...] = x


def kernel(query, context, W_enc, b_enc, Wp, bp, gp, betap):
    del context
    q2 = query.reshape(1, _D_IN)
    be2 = b_enc.reshape(1, _D_H)
    out = pl.pallas_call(
        _body,
        in_specs=[
            pl.BlockSpec(memory_space=pltpu.MemorySpace.VMEM),
            pl.BlockSpec(memory_space=pltpu.MemorySpace.HBM),
            pl.BlockSpec(memory_space=pltpu.MemorySpace.VMEM),
            pl.BlockSpec(memory_space=pltpu.MemorySpace.HBM),
            pl.BlockSpec(memory_space=pltpu.MemorySpace.VMEM),
            pl.BlockSpec(memory_space=pltpu.MemorySpace.VMEM),
            pl.BlockSpec(memory_space=pltpu.MemorySpace.VMEM),
        ],
        out_specs=pl.BlockSpec(memory_space=pltpu.MemorySpace.VMEM),
        out_shape=jax.ShapeDtypeStruct((1, _D_H), jnp.float32),
        scratch_shapes=[
            pltpu.VMEM((_D_H, _D_IN), jnp.float32),
            pltpu.VMEM((_D_H, _D_H), jnp.float32),
            pltpu.VMEM((_D_H, _D_H), jnp.float32),
            pltpu.VMEM((_D_H, _D_H), jnp.float32),
            pltpu.SemaphoreType.DMA,
            pltpu.SemaphoreType.DMA((3, 2)),
        ],
    )(q2, W_enc, be2, Wp, bp.reshape(_NL, 1, _D_H), gp.reshape(_NL, 1, _D_H),
      betap.reshape(_NL, 1, _D_H))
    return out.reshape(_D_H)


# speedup vs baseline: 1.1746x; 1.0621x over previous
import jax
import jax.numpy as jnp
from jax.experimental import pallas as pl
from jax.experimental.pallas import tpu as pltpu

_D_IN = 768
_D_H = 1024
_NL = 4


def _matvec(x, w):
    return jax.lax.dot_general(
        x.astype(jnp.bfloat16), w.astype(jnp.bfloat16),
        (((1,), (1,)), ((), ())),
        preferred_element_type=jnp.float32)


def _body(q_ref, we_hbm, be_ref, wp_hbm, bp_ref, gp_ref, betap_ref,
          out_ref, we_v, wb0, wb1, wb2, sem_we, sem_w):
    wbufs = [wb0, wb1, wb2]
    cp_we = pltpu.make_async_copy(we_hbm, we_v, sem_we)
    cp_we.start()
    for i in range(2):
        pltpu.make_async_copy(wp_hbm.at[i], wbufs[i], sem_w.at[i]).start()

    cp_we.wait()
    h = _matvec(q_ref[...], we_v[...]) + be_ref[...]
    n = jnp.sqrt(jnp.sum(h * h))
    x = h / jnp.maximum(n, 1e-12)

    for i in range(_NL):
        if i + 2 < _NL:
            pltpu.make_async_copy(
                wp_hbm.at[i + 2], wbufs[(i + 2) % 3],
                sem_w.at[(i + 2) % 3]).start()
        pltpu.make_async_copy(
            wp_hbm.at[i], wbufs[i % 3], sem_w.at[i % 3]).wait()
        h = _matvec(x, wbufs[i % 3][...]) + bp_ref[i, 0][None]
        h = 0.5 * h * (1.0 + jax.lax.erf(h * 0.7071067811865476))
        mu = jnp.mean(h, axis=-1, keepdims=True)
        var = jnp.mean((h - mu) * (h - mu), axis=-1, keepdims=True)
        h = (h - mu) / jnp.sqrt(var + 1e-5) * gp_ref[i, 0][None] \
            + betap_ref[i, 0][None]
        x = x + h

    out_ref[...] = x


def kernel(query, context, W_enc, b_enc, Wp, bp, gp, betap):
    del context
    q2 = query.reshape(1, _D_IN)
    be2 = b_enc.reshape(1, _D_H)
    out = pl.pallas_call(
        _body,
        in_specs=[
            pl.BlockSpec(memory_space=pltpu.MemorySpace.VMEM),
            pl.BlockSpec(memory_space=pltpu.MemorySpace.HBM),
            pl.BlockSpec(memory_space=pltpu.MemorySpace.VMEM),
            pl.BlockSpec(memory_space=pltpu.MemorySpace.HBM),
            pl.BlockSpec(memory_space=pltpu.MemorySpace.VMEM),
            pl.BlockSpec(memory_space=pltpu.MemorySpace.VMEM),
            pl.BlockSpec(memory_space=pltpu.MemorySpace.VMEM),
        ],
        out_specs=pl.BlockSpec(memory_space=pltpu.MemorySpace.VMEM),
        out_shape=jax.ShapeDtypeStruct((1, _D_H), jnp.float32),
        scratch_shapes=[
            pltpu.VMEM((_D_H, _D_IN), jnp.float32),
            pltpu.VMEM((_D_H, _D_H), jnp.float32),
            pltpu.VMEM((_D_H, _D_H), jnp.float32),
            pltpu.VMEM((_D_H, _D_H), jnp.float32),
            pltpu.SemaphoreType.DMA,
            pltpu.SemaphoreType.DMA((3,)),
        ],
    )(q2, W_enc, be2, Wp, bp.reshape(_NL, 1, _D_H), gp.reshape(_NL, 1, _D_H),
      betap.reshape(_NL, 1, _D_H))
    return out.reshape(_D_H)
